# baseline (speedup 1.0000x reference)
import jax
import jax.numpy as jnp
from jax import lax
from jax.experimental import pallas as pl

FEATS_ = 128
K_ = 50000
BLOCK_ = 5000


def _scale_kernel(x_ref, w_ref, o_ref):
    x = x_ref[...]
    w = w_ref[...]
    inv_norm = jax.lax.rsqrt(jnp.sum(w * w))
    s = jnp.dot(x, w, preferred_element_type=jnp.float32) * inv_norm
    o_ref[...] = x * jnp.tanh(s)


def kernel(node_embs, mask, scorer):
    del mask
    n_blocks = pl.cdiv(K_, BLOCK_)
    out = pl.pallas_call(
        _scale_kernel,
        grid=(n_blocks,),
        in_specs=[
            pl.BlockSpec((BLOCK_, FEATS_), lambda i: (i, 0)),
            pl.BlockSpec((FEATS_, 1), lambda i: (0, 0)),
        ],
        out_specs=pl.BlockSpec((BLOCK_, FEATS_), lambda i: (i, 0)),
        out_shape=jax.ShapeDtypeStruct((K_, FEATS_), jnp.float32),
    )(node_embs, scorer)
    return out.T


# speedup vs baseline: 1.0956x; 1.0956x over previous
import jax
import jax.numpy as jnp
from jax import lax
from jax.experimental import pallas as pl

FEATS_ = 128
K_ = 50000
BLOCK_ = 12504


def _scale_kernel(x_ref, w_ref, o_ref):
    x = x_ref[...]
    w = w_ref[...]
    inv_norm = jax.lax.rsqrt(jnp.sum(w * w))
    s = jnp.dot(x, w, preferred_element_type=jnp.float32) * inv_norm
    o_ref[...] = x * jnp.tanh(s)


def kernel(node_embs, mask, scorer):
    del mask
    n_blocks = pl.cdiv(K_, BLOCK_)
    out = pl.pallas_call(
        _scale_kernel,
        grid=(n_blocks,),
        in_specs=[
            pl.BlockSpec((BLOCK_, FEATS_), lambda i: (i, 0)),
            pl.BlockSpec((FEATS_, 1), lambda i: (0, 0)),
        ],
        out_specs=pl.BlockSpec((BLOCK_, FEATS_), lambda i: (i, 0)),
        out_shape=jax.ShapeDtypeStruct((K_, FEATS_), jnp.float32),
    )(node_embs, scorer)
    return out.T


# speedup vs baseline: 1.1021x; 1.0059x over previous
import jax
import jax.numpy as jnp
from jax import lax
from jax.experimental import pallas as pl

FEATS_ = 128
K_ = 50000
BLOCK_ = 16672


def _scale_kernel(x_ref, w_ref, o_ref):
    x = x_ref[...]
    w = w_ref[...]
    inv_norm = jax.lax.rsqrt(jnp.sum(w * w))
    s = jnp.dot(x, w, preferred_element_type=jnp.float32) * inv_norm
    o_ref[...] = x * jnp.tanh(s)


def kernel(node_embs, mask, scorer):
    del mask
    n_blocks = pl.cdiv(K_, BLOCK_)
    out = pl.pallas_call(
        _scale_kernel,
        grid=(n_blocks,),
        in_specs=[
            pl.BlockSpec((BLOCK_, FEATS_), lambda i: (i, 0)),
            pl.BlockSpec((FEATS_, 1), lambda i: (0, 0)),
        ],
        out_specs=pl.BlockSpec((BLOCK_, FEATS_), lambda i: (i, 0)),
        out_shape=jax.ShapeDtypeStruct((K_, FEATS_), jnp.float32),
    )(node_embs, scorer)
    return out.T
